# Initial kernel scaffold; baseline (speedup 1.0000x reference)
import jax
import jax.numpy as jnp
from jax.experimental import pallas as pl


def kernel(input_ids, anchor):
    raise NotImplementedError("write your pallas kernel here")


# speedup vs baseline: 1.1483x; 1.1483x over previous
import jax
import jax.numpy as jnp
from jax.experimental import pallas as pl
from jax.experimental.pallas import tpu as pltpu

_VOCAB = 32000


def _body(ids_ref, anchor_ref, out_ref):
    b = pl.program_id(0)
    s = out_ref.shape[1]
    tok = (ids_ref[b, s - 1] + 1) % (_VOCAB - 1) + 1
    val = 10.0 + anchor_ref[0]
    out_ref[...] = jnp.zeros(out_ref.shape, jnp.float32)
    col = jax.lax.broadcasted_iota(jnp.int32, (1, _VOCAB), 1)
    out_ref[:, s - 1, :] = jnp.where(col == tok, val, 0.0)


def kernel(input_ids, anchor):
    batch, seq_len = input_ids.shape
    grid_spec = pltpu.PrefetchScalarGridSpec(
        num_scalar_prefetch=2,
        grid=(batch,),
        in_specs=[],
        out_specs=pl.BlockSpec(
            (1, seq_len, _VOCAB), lambda b, ids, anc: (b, 0, 0)
        ),
    )
    return pl.pallas_call(
        _body,
        grid_spec=grid_spec,
        out_shape=jax.ShapeDtypeStruct((batch, seq_len, _VOCAB), jnp.float32),
    )(input_ids, anchor)
